# initial kernel scaffold (unmeasured)
import jax
import jax.numpy as jnp
from jax import lax
from jax.experimental import pallas as pl
from jax.experimental.pallas import tpu as pltpu


def kernel(partial, gamma):
    _, m_global, d = partial.shape
    m_half = m_global // 2

    def body(partial_ref, gamma_ref, out_ref, comm_ref, send_sem, recv_sem):
        my_x = lax.axis_index("x")
        my_y = lax.axis_index("y")
        my_z = lax.axis_index("z")
        partner = (my_x, my_y, 1 - my_z)

        barrier_sem = pltpu.get_barrier_semaphore()
        pl.semaphore_signal(
            barrier_sem, inc=1,
            device_id=partner, device_id_type=pl.DeviceIdType.MESH,
        )
        pl.semaphore_wait(barrier_sem, 1)

        rdma = pltpu.make_async_remote_copy(
            src_ref=partial_ref.at[0, pl.ds((1 - my_z) * m_half, m_half), :],
            dst_ref=comm_ref,
            send_sem=send_sem,
            recv_sem=recv_sem,
            device_id=partner,
            device_id_type=pl.DeviceIdType.MESH,
        )
        rdma.start()
        rdma.wait()

        y = partial_ref[0, pl.ds(my_z * m_half, m_half), :] + comm_ref[:, :]
        rms = jnp.sqrt(jnp.mean(y * y, axis=-1, keepdims=True) + 1e-6)
        out_ref[:, :] = (y / rms) * gamma_ref[:][None, :]

    return pl.pallas_call(
        body,
        out_shape=jax.ShapeDtypeStruct((m_half, d), jnp.float32),
        in_specs=[
            pl.BlockSpec(memory_space=pltpu.VMEM),
            pl.BlockSpec(memory_space=pltpu.VMEM),
        ],
        out_specs=pl.BlockSpec(memory_space=pltpu.VMEM),
        scratch_shapes=[
            pltpu.VMEM((m_half, d), jnp.float32),
            pltpu.SemaphoreType.DMA,
            pltpu.SemaphoreType.DMA,
        ],
        compiler_params=pltpu.CompilerParams(collective_id=0),
    )(partial, gamma)


# baseline (device time: 206457 ns/iter reference)
import jax
import jax.numpy as jnp
from jax import lax
from jax.experimental import pallas as pl
from jax.experimental.pallas import tpu as pltpu

CHUNK = 256


def kernel(partial, gamma):
    _, m_global, d = partial.shape
    m_half = m_global // 2
    n_chunks = m_half // CHUNK

    def body(partial_ref, gamma_ref, out_ref, comm_ref, local_buf,
             send_sem, recv_sem, copy_sem):
        my_x = lax.axis_index("x")
        my_y = lax.axis_index("y")
        my_z = lax.axis_index("z")
        partner = (my_x, my_y, 1 - my_z)

        barrier_sem = pltpu.get_barrier_semaphore()
        pl.semaphore_signal(
            barrier_sem, inc=1,
            device_id=partner, device_id_type=pl.DeviceIdType.MESH,
        )
        pl.semaphore_wait(barrier_sem, 1)

        rdma = pltpu.make_async_remote_copy(
            src_ref=partial_ref.at[0, pl.ds((1 - my_z) * m_half, m_half), :],
            dst_ref=comm_ref,
            send_sem=send_sem,
            recv_sem=recv_sem,
            device_id=partner,
            device_id_type=pl.DeviceIdType.MESH,
        )
        rdma.start()
        rdma.wait()

        for c in range(n_chunks):
            row0 = c * CHUNK
            cp = pltpu.make_async_copy(
                partial_ref.at[0, pl.ds(my_z * m_half + row0, CHUNK), :],
                local_buf,
                copy_sem,
            )
            cp.start()
            cp.wait()
            y = local_buf[:, :] + comm_ref[pl.ds(row0, CHUNK), :]
            rms = jnp.sqrt(jnp.mean(y * y, axis=-1, keepdims=True) + 1e-6)
            out_ref[pl.ds(row0, CHUNK), :] = (y / rms) * gamma_ref[:][None, :]

    return pl.pallas_call(
        body,
        out_shape=jax.ShapeDtypeStruct((m_half, d), jnp.float32),
        in_specs=[
            pl.BlockSpec(memory_space=pl.ANY),
            pl.BlockSpec(memory_space=pltpu.VMEM),
        ],
        out_specs=pl.BlockSpec(memory_space=pltpu.VMEM),
        scratch_shapes=[
            pltpu.VMEM((m_half, d), jnp.float32),
            pltpu.VMEM((CHUNK, d), jnp.float32),
            pltpu.SemaphoreType.DMA,
            pltpu.SemaphoreType.DMA,
            pltpu.SemaphoreType.DMA,
        ],
        compiler_params=pltpu.CompilerParams(collective_id=0),
    )(partial, gamma)


# device time: 204715 ns/iter; 1.0085x vs baseline; 1.0085x over previous
import jax
import jax.numpy as jnp
from jax import lax
from jax.experimental import pallas as pl
from jax.experimental.pallas import tpu as pltpu

NCHUNK = 8


def kernel(partial, gamma):
    _, m_global, d = partial.shape
    m_half = m_global // 2
    rows = m_half // NCHUNK

    def body(partial_ref, gamma_ref, out_ref, comm_ref, local_ref,
             send_sems, recv_sems, copy_sem):
        my_x = lax.axis_index("x")
        my_y = lax.axis_index("y")
        my_z = lax.axis_index("z")
        partner = (my_x, my_y, 1 - my_z)

        barrier_sem = pltpu.get_barrier_semaphore()
        pl.semaphore_signal(
            barrier_sem, inc=1,
            device_id=partner, device_id_type=pl.DeviceIdType.MESH,
        )
        pl.semaphore_wait(barrier_sem, 1)

        local_cp = pltpu.make_async_copy(
            partial_ref.at[0, pl.ds(my_z * m_half, m_half), :],
            local_ref,
            copy_sem,
        )
        local_cp.start()

        rdmas = []
        for c in range(NCHUNK):
            rdma = pltpu.make_async_remote_copy(
                src_ref=partial_ref.at[
                    0, pl.ds((1 - my_z) * m_half + c * rows, rows), :
                ],
                dst_ref=comm_ref.at[pl.ds(c * rows, rows), :],
                send_sem=send_sems.at[c],
                recv_sem=recv_sems.at[c],
                device_id=partner,
                device_id_type=pl.DeviceIdType.MESH,
            )
            rdma.start()
            rdmas.append(rdma)

        local_cp.wait()

        for c in range(NCHUNK):
            rdmas[c].wait_recv()
            row0 = c * rows
            y = (local_ref[pl.ds(row0, rows), :]
                 + comm_ref[pl.ds(row0, rows), :])
            rms = jnp.sqrt(jnp.mean(y * y, axis=-1, keepdims=True) + 1e-6)
            out_ref[pl.ds(row0, rows), :] = (y / rms) * gamma_ref[:][None, :]

        for c in range(NCHUNK):
            rdmas[c].wait_send()

    return pl.pallas_call(
        body,
        out_shape=jax.ShapeDtypeStruct((m_half, d), jnp.float32),
        in_specs=[
            pl.BlockSpec(memory_space=pl.ANY),
            pl.BlockSpec(memory_space=pltpu.VMEM),
        ],
        out_specs=pl.BlockSpec(memory_space=pltpu.VMEM),
        scratch_shapes=[
            pltpu.VMEM((m_half, d), jnp.float32),
            pltpu.VMEM((m_half, d), jnp.float32),
            pltpu.SemaphoreType.DMA((NCHUNK,)),
            pltpu.SemaphoreType.DMA((NCHUNK,)),
            pltpu.SemaphoreType.DMA,
        ],
        compiler_params=pltpu.CompilerParams(
            collective_id=0,
            vmem_limit_bytes=56 * 1024 * 1024,
        ),
    )(partial, gamma)


# device time: 106937 ns/iter; 1.9306x vs baseline; 1.9144x over previous
import jax
import jax.numpy as jnp
from jax import lax
from jax.experimental import pallas as pl
from jax.experimental.pallas import tpu as pltpu

M_HALF = 2048
QROWS = 512
CH = 128
NCQ = QROWS // CH
NF = NCQ // 2


def kernel(partial, gamma):
    _, m_global, d = partial.shape
    assert m_global // 2 == M_HALF

    def body(partial_ref, gamma_ref, out_ref, comm_ref, local_ref,
             zsend, zrecv, dsendL, dsendR, drecvL, drecvR,
             fsendL, fsendR, frecvL, frecvR, copy_sem):
        my_x = lax.axis_index("x")
        my_y = lax.axis_index("y")
        my_z = lax.axis_index("z")
        partner = (my_x, my_y, 1 - my_z)

        p = jnp.where(my_x == 0, my_y, 3 - my_y)

        def pos_xy(q):
            q = q % 4
            return q // 2, ((q + 1) % 4) // 2

        lx, ly = pos_xy(p - 1)
        rx, ry = pos_xy(p + 1)
        left = (lx, ly, my_z)
        right = (rx, ry, my_z)

        def rows(qi, c):
            return (qi % 4) * QROWS + c * CH

        local_cp = pltpu.make_async_copy(
            partial_ref.at[0, pl.ds(my_z * M_HALF, M_HALF), :],
            local_ref,
            copy_sem,
        )
        local_cp.start()

        barrier_sem = pltpu.get_barrier_semaphore()
        for nbr in (partner, left, right):
            pl.semaphore_signal(
                barrier_sem, inc=1,
                device_id=nbr, device_id_type=pl.DeviceIdType.MESH,
            )
        pl.semaphore_wait(barrier_sem, 3)

        z_send = [
            pltpu.make_async_remote_copy(
                src_ref=partial_ref.at[
                    0, pl.ds((1 - my_z) * M_HALF + rows(p, c), CH), :
                ],
                dst_ref=comm_ref.at[pl.ds(rows(p, c), CH), :],
                send_sem=zsend.at[c],
                recv_sem=zrecv.at[c],
                device_id=partner,
                device_id_type=pl.DeviceIdType.MESH,
            )
            for c in range(NCQ)
        ]
        d_sendR = [
            pltpu.make_async_remote_copy(
                src_ref=comm_ref.at[pl.ds(rows(p, c), CH), :],
                dst_ref=comm_ref.at[pl.ds(rows(p, c), CH), :],
                send_sem=dsendR.at[c],
                recv_sem=drecvL.at[c],
                device_id=right,
                device_id_type=pl.DeviceIdType.MESH,
            )
            for c in range(NCQ)
        ]
        d_sendL = [
            pltpu.make_async_remote_copy(
                src_ref=comm_ref.at[pl.ds(rows(p, c), CH), :],
                dst_ref=comm_ref.at[pl.ds(rows(p, c), CH), :],
                send_sem=dsendL.at[c],
                recv_sem=drecvR.at[c],
                device_id=left,
                device_id_type=pl.DeviceIdType.MESH,
            )
            for c in range(NCQ)
        ]
        f_sendR = [
            pltpu.make_async_remote_copy(
                src_ref=comm_ref.at[pl.ds(rows(p - 1, c), CH), :],
                dst_ref=comm_ref.at[pl.ds(rows(p - 1, c), CH), :],
                send_sem=fsendR.at[c - NF],
                recv_sem=frecvL.at[c - NF],
                device_id=right,
                device_id_type=pl.DeviceIdType.MESH,
            )
            for c in range(NF, NCQ)
        ]
        f_sendL = [
            pltpu.make_async_remote_copy(
                src_ref=comm_ref.at[pl.ds(rows(p + 1, c), CH), :],
                dst_ref=comm_ref.at[pl.ds(rows(p + 1, c), CH), :],
                send_sem=fsendL.at[c],
                recv_sem=frecvR.at[c],
                device_id=left,
                device_id_type=pl.DeviceIdType.MESH,
            )
            for c in range(NF)
        ]

        def recv_only(qi, c, sem):
            return pltpu.make_async_remote_copy(
                src_ref=comm_ref.at[pl.ds(rows(qi, c), CH), :],
                dst_ref=comm_ref.at[pl.ds(rows(qi, c), CH), :],
                send_sem=zsend.at[0],
                recv_sem=sem,
                device_id=partner,
                device_id_type=pl.DeviceIdType.MESH,
            )

        z_recv = [recv_only(p, c, zrecv.at[c]) for c in range(NCQ)]
        dL_recv = [recv_only(p - 1, c, drecvL.at[c]) for c in range(NCQ)]
        dR_recv = [recv_only(p + 1, c, drecvR.at[c]) for c in range(NCQ)]
        fL_recv = [recv_only(p + 2, c + NF, frecvL.at[c]) for c in range(NF)]
        fR_recv = [recv_only(p + 2, c, frecvR.at[c]) for c in range(NF)]

        def compute_rows(r0):
            y = local_ref[pl.ds(r0, CH), :] + comm_ref[pl.ds(r0, CH), :]
            rms = jnp.sqrt(jnp.mean(y * y, axis=-1, keepdims=True) + 1e-6)
            out_ref[pl.ds(r0, CH), :] = (y / rms) * gamma_ref[:][None, :]

        for c in range(NCQ):
            z_send[c].start()
        local_cp.wait()

        for c in range(NCQ):
            z_recv[c].wait_recv()
            d_sendR[c].start()
            d_sendL[c].start()
            compute_rows(rows(p, c))

        for c in range(NCQ):
            dL_recv[c].wait_recv()
            if c >= NF:
                f_sendR[c - NF].start()
            compute_rows(rows(p - 1, c))
            dR_recv[c].wait_recv()
            if c < NF:
                f_sendL[c].start()
            compute_rows(rows(p + 1, c))

        for c in range(NF):
            fR_recv[c].wait_recv()
            compute_rows(rows(p + 2, c))
            fL_recv[c].wait_recv()
            compute_rows(rows(p + 2, c + NF))

        for c in range(NCQ):
            z_send[c].wait_send()
            d_sendR[c].wait_send()
            d_sendL[c].wait_send()
        for c in range(NF):
            f_sendR[c].wait_send()
            f_sendL[c].wait_send()

    return pl.pallas_call(
        body,
        out_shape=jax.ShapeDtypeStruct((M_HALF, d), jnp.float32),
        in_specs=[
            pl.BlockSpec(memory_space=pl.ANY),
            pl.BlockSpec(memory_space=pltpu.VMEM),
        ],
        out_specs=pl.BlockSpec(memory_space=pltpu.VMEM),
        scratch_shapes=[
            pltpu.VMEM((M_HALF, d), jnp.float32),
            pltpu.VMEM((M_HALF, d), jnp.float32),
            pltpu.SemaphoreType.DMA((NCQ,)),
            pltpu.SemaphoreType.DMA((NCQ,)),
            pltpu.SemaphoreType.DMA((NCQ,)),
            pltpu.SemaphoreType.DMA((NCQ,)),
            pltpu.SemaphoreType.DMA((NCQ,)),
            pltpu.SemaphoreType.DMA((NCQ,)),
            pltpu.SemaphoreType.DMA((NF,)),
            pltpu.SemaphoreType.DMA((NF,)),
            pltpu.SemaphoreType.DMA((NF,)),
            pltpu.SemaphoreType.DMA((NF,)),
            pltpu.SemaphoreType.DMA,
        ],
        compiler_params=pltpu.CompilerParams(
            collective_id=0,
            vmem_limit_bytes=56 * 1024 * 1024,
        ),
    )(partial, gamma)


# device time: 102339 ns/iter; 2.0174x vs baseline; 1.0449x over previous
import jax
import jax.numpy as jnp
from jax import lax
from jax.experimental import pallas as pl
from jax.experimental.pallas import tpu as pltpu

M_HALF = 2048
QROWS = 512
CH = 128
NCQ = QROWS // CH
NF = NCQ // 2


def kernel(partial, gamma):
    _, m_global, d = partial.shape
    assert m_global // 2 == M_HALF

    def body(partial_ref, gamma_ref, out_ref, comm_ref, local_ref, stage_ref,
             zsend, zrecv, dsendL, dsendR, drecvL, drecvR,
             fsendL, fsendR, frecvL, frecvR, copy_sem, osem):
        my_x = lax.axis_index("x")
        my_y = lax.axis_index("y")
        my_z = lax.axis_index("z")
        partner = (my_x, my_y, 1 - my_z)

        p = jnp.where(my_x == 0, my_y, 3 - my_y)

        def pos_xy(q):
            q = q % 4
            return q // 2, ((q + 1) % 4) // 2

        lx, ly = pos_xy(p - 1)
        rx, ry = pos_xy(p + 1)
        left = (lx, ly, my_z)
        right = (rx, ry, my_z)

        def rows(qi, c):
            return (qi % 4) * QROWS + c * CH

        local_cp = pltpu.make_async_copy(
            partial_ref.at[0, pl.ds(my_z * M_HALF, M_HALF), :],
            local_ref,
            copy_sem,
        )
        local_cp.start()

        barrier_sem = pltpu.get_barrier_semaphore()
        for nbr in (partner, left, right):
            pl.semaphore_signal(
                barrier_sem, inc=1,
                device_id=nbr, device_id_type=pl.DeviceIdType.MESH,
            )
        pl.semaphore_wait(barrier_sem, 3)

        z_send = [
            pltpu.make_async_remote_copy(
                src_ref=partial_ref.at[
                    0, pl.ds((1 - my_z) * M_HALF + rows(p, c), CH), :
                ],
                dst_ref=comm_ref.at[pl.ds(rows(p, c), CH), :],
                send_sem=zsend.at[c],
                recv_sem=zrecv.at[c],
                device_id=partner,
                device_id_type=pl.DeviceIdType.MESH,
            )
            for c in range(NCQ)
        ]
        d_sendR = [
            pltpu.make_async_remote_copy(
                src_ref=comm_ref.at[pl.ds(rows(p, c), CH), :],
                dst_ref=comm_ref.at[pl.ds(rows(p, c), CH), :],
                send_sem=dsendR.at[c],
                recv_sem=drecvL.at[c],
                device_id=right,
                device_id_type=pl.DeviceIdType.MESH,
            )
            for c in range(NCQ)
        ]
        d_sendL = [
            pltpu.make_async_remote_copy(
                src_ref=comm_ref.at[pl.ds(rows(p, c), CH), :],
                dst_ref=comm_ref.at[pl.ds(rows(p, c), CH), :],
                send_sem=dsendL.at[c],
                recv_sem=drecvR.at[c],
                device_id=left,
                device_id_type=pl.DeviceIdType.MESH,
            )
            for c in range(NCQ)
        ]
        f_sendR = [
            pltpu.make_async_remote_copy(
                src_ref=comm_ref.at[pl.ds(rows(p - 1, c), CH), :],
                dst_ref=comm_ref.at[pl.ds(rows(p - 1, c), CH), :],
                send_sem=fsendR.at[c - NF],
                recv_sem=frecvL.at[c - NF],
                device_id=right,
                device_id_type=pl.DeviceIdType.MESH,
            )
            for c in range(NF, NCQ)
        ]
        f_sendL = [
            pltpu.make_async_remote_copy(
                src_ref=comm_ref.at[pl.ds(rows(p + 1, c), CH), :],
                dst_ref=comm_ref.at[pl.ds(rows(p + 1, c), CH), :],
                send_sem=fsendL.at[c],
                recv_sem=frecvR.at[c],
                device_id=left,
                device_id_type=pl.DeviceIdType.MESH,
            )
            for c in range(NF)
        ]

        def recv_only(qi, c, sem):
            return pltpu.make_async_remote_copy(
                src_ref=comm_ref.at[pl.ds(rows(qi, c), CH), :],
                dst_ref=comm_ref.at[pl.ds(rows(qi, c), CH), :],
                send_sem=zsend.at[0],
                recv_sem=sem,
                device_id=partner,
                device_id_type=pl.DeviceIdType.MESH,
            )

        z_recv = [recv_only(p, c, zrecv.at[c]) for c in range(NCQ)]
        dL_recv = [recv_only(p - 1, c, drecvL.at[c]) for c in range(NCQ)]
        dR_recv = [recv_only(p + 1, c, drecvR.at[c]) for c in range(NCQ)]
        fL_recv = [recv_only(p + 2, c + NF, frecvL.at[c]) for c in range(NF)]
        fR_recv = [recv_only(p + 2, c, frecvR.at[c]) for c in range(NF)]

        out_cps = []

        def compute_rows(r0):
            y = local_ref[pl.ds(r0, CH), :] + comm_ref[pl.ds(r0, CH), :]
            rms = jnp.sqrt(jnp.mean(y * y, axis=-1, keepdims=True) + 1e-6)
            stage_ref[pl.ds(r0, CH), :] = (y / rms) * gamma_ref[:][None, :]
            cp = pltpu.make_async_copy(
                stage_ref.at[pl.ds(r0, CH), :],
                out_ref.at[pl.ds(r0, CH), :],
                osem.at[len(out_cps)],
            )
            cp.start()
            out_cps.append(cp)

        for c in range(NCQ):
            z_send[c].start()
        local_cp.wait()

        for c in range(NCQ):
            z_recv[c].wait_recv()
            d_sendR[c].start()
            d_sendL[c].start()
            compute_rows(rows(p, c))

        for c in range(NCQ):
            dL_recv[c].wait_recv()
            if c >= NF:
                f_sendR[c - NF].start()
            compute_rows(rows(p - 1, c))
            dR_recv[c].wait_recv()
            if c < NF:
                f_sendL[c].start()
            compute_rows(rows(p + 1, c))

        for c in range(NF):
            fR_recv[c].wait_recv()
            compute_rows(rows(p + 2, c))
            fL_recv[c].wait_recv()
            compute_rows(rows(p + 2, c + NF))

        for c in range(NCQ):
            z_send[c].wait_send()
            d_sendR[c].wait_send()
            d_sendL[c].wait_send()
        for c in range(NF):
            f_sendR[c].wait_send()
            f_sendL[c].wait_send()
        for cp in out_cps:
            cp.wait()

    return pl.pallas_call(
        body,
        out_shape=jax.ShapeDtypeStruct((M_HALF, d), jnp.float32),
        in_specs=[
            pl.BlockSpec(memory_space=pl.ANY),
            pl.BlockSpec(memory_space=pltpu.VMEM),
        ],
        out_specs=pl.BlockSpec(memory_space=pl.ANY),
        scratch_shapes=[
            pltpu.VMEM((M_HALF, d), jnp.float32),
            pltpu.VMEM((M_HALF, d), jnp.float32),
            pltpu.VMEM((M_HALF, d), jnp.float32),
            pltpu.SemaphoreType.DMA((NCQ,)),
            pltpu.SemaphoreType.DMA((NCQ,)),
            pltpu.SemaphoreType.DMA((NCQ,)),
            pltpu.SemaphoreType.DMA((NCQ,)),
            pltpu.SemaphoreType.DMA((NCQ,)),
            pltpu.SemaphoreType.DMA((NCQ,)),
            pltpu.SemaphoreType.DMA((NF,)),
            pltpu.SemaphoreType.DMA((NF,)),
            pltpu.SemaphoreType.DMA((NF,)),
            pltpu.SemaphoreType.DMA((NF,)),
            pltpu.SemaphoreType.DMA,
            pltpu.SemaphoreType.DMA((4 * NCQ,)),
        ],
        compiler_params=pltpu.CompilerParams(
            collective_id=0,
            vmem_limit_bytes=56 * 1024 * 1024,
        ),
    )(partial, gamma)


# device time: 92510 ns/iter; 2.2317x vs baseline; 1.1062x over previous
import jax
import jax.numpy as jnp
from jax import lax
from jax.experimental import pallas as pl
from jax.experimental.pallas import tpu as pltpu

M_HALF = 2048
DROWS = 256
DCH = 128
NCD = DROWS // DCH
QR = 448
QCH = 112
NCQ = QR // QCH
NF = NCQ // 2
QBASE = DROWS
assert QBASE + 4 * QR == M_HALF


def kernel(partial, gamma):
    _, m_global, d = partial.shape
    assert m_global // 2 == M_HALF

    def body(partial_ref, gamma_ref, out_ref, comm_ref, local_ref, stage_ref,
             zqsend, zqrecv, zdsend, zdrecv,
             dsendL, dsendR, drecvL, drecvR,
             fsendL, fsendR, frecvL, frecvR, copy_sem, osem):
        my_x = lax.axis_index("x")
        my_y = lax.axis_index("y")
        my_z = lax.axis_index("z")
        partner = (my_x, my_y, 1 - my_z)

        p = jnp.where(my_x == 0, my_y, 3 - my_y)

        def pos_xy(q):
            q = q % 4
            return q // 2, ((q + 1) % 4) // 2

        lx, ly = pos_xy(p - 1)
        rx, ry = pos_xy(p + 1)
        left = (lx, ly, my_z)
        right = (rx, ry, my_z)

        def rows_q(qi, c):
            return QBASE + (qi % 4) * QR + c * QCH

        local_cp = pltpu.make_async_copy(
            partial_ref.at[0, pl.ds(my_z * M_HALF, M_HALF), :],
            local_ref,
            copy_sem,
        )
        local_cp.start()

        barrier_sem = pltpu.get_barrier_semaphore()
        for nbr in (partner, left, right):
            pl.semaphore_signal(
                barrier_sem, inc=1,
                device_id=nbr, device_id_type=pl.DeviceIdType.MESH,
            )
        pl.semaphore_wait(barrier_sem, 3)

        zq_send = [
            pltpu.make_async_remote_copy(
                src_ref=partial_ref.at[
                    0, pl.ds((1 - my_z) * M_HALF + rows_q(p, c), QCH), :
                ],
                dst_ref=comm_ref.at[pl.ds(rows_q(p, c), QCH), :],
                send_sem=zqsend.at[c],
                recv_sem=zqrecv.at[c],
                device_id=partner,
                device_id_type=pl.DeviceIdType.MESH,
            )
            for c in range(NCQ)
        ]
        zd_send = [
            pltpu.make_async_remote_copy(
                src_ref=partial_ref.at[
                    0, pl.ds((1 - my_z) * M_HALF + c * DCH, DCH), :
                ],
                dst_ref=comm_ref.at[pl.ds(c * DCH, DCH), :],
                send_sem=zdsend.at[c],
                recv_sem=zdrecv.at[c],
                device_id=partner,
                device_id_type=pl.DeviceIdType.MESH,
            )
            for c in range(NCD)
        ]
        d_sendR = [
            pltpu.make_async_remote_copy(
                src_ref=comm_ref.at[pl.ds(rows_q(p, c), QCH), :],
                dst_ref=comm_ref.at[pl.ds(rows_q(p, c), QCH), :],
                send_sem=dsendR.at[c],
                recv_sem=drecvL.at[c],
                device_id=right,
                device_id_type=pl.DeviceIdType.MESH,
            )
            for c in range(NCQ)
        ]
        d_sendL = [
            pltpu.make_async_remote_copy(
                src_ref=comm_ref.at[pl.ds(rows_q(p, c), QCH), :],
                dst_ref=comm_ref.at[pl.ds(rows_q(p, c), QCH), :],
                send_sem=dsendL.at[c],
                recv_sem=drecvR.at[c],
                device_id=left,
                device_id_type=pl.DeviceIdType.MESH,
            )
            for c in range(NCQ)
        ]
        f_sendR = [
            pltpu.make_async_remote_copy(
                src_ref=comm_ref.at[pl.ds(rows_q(p - 1, c), QCH), :],
                dst_ref=comm_ref.at[pl.ds(rows_q(p - 1, c), QCH), :],
                send_sem=fsendR.at[c - NF],
                recv_sem=frecvL.at[c - NF],
                device_id=right,
                device_id_type=pl.DeviceIdType.MESH,
            )
            for c in range(NF, NCQ)
        ]
        f_sendL = [
            pltpu.make_async_remote_copy(
                src_ref=comm_ref.at[pl.ds(rows_q(p + 1, c), QCH), :],
                dst_ref=comm_ref.at[pl.ds(rows_q(p + 1, c), QCH), :],
                send_sem=fsendL.at[c],
                recv_sem=frecvR.at[c],
                device_id=left,
                device_id_type=pl.DeviceIdType.MESH,
            )
            for c in range(NF)
        ]

        def recv_only(r0, n, sem):
            return pltpu.make_async_remote_copy(
                src_ref=comm_ref.at[pl.ds(r0, n), :],
                dst_ref=comm_ref.at[pl.ds(r0, n), :],
                send_sem=zqsend.at[0],
                recv_sem=sem,
                device_id=partner,
                device_id_type=pl.DeviceIdType.MESH,
            )

        zq_recv = [recv_only(rows_q(p, c), QCH, zqrecv.at[c])
                   for c in range(NCQ)]
        zd_recv = [recv_only(c * DCH, DCH, zdrecv.at[c]) for c in range(NCD)]
        dL_recv = [recv_only(rows_q(p - 1, c), QCH, drecvL.at[c])
                   for c in range(NCQ)]
        dR_recv = [recv_only(rows_q(p + 1, c), QCH, drecvR.at[c])
                   for c in range(NCQ)]
        fL_recv = [recv_only(rows_q(p + 2, c + NF), QCH, frecvL.at[c])
                   for c in range(NF)]
        fR_recv = [recv_only(rows_q(p + 2, c), QCH, frecvR.at[c])
                   for c in range(NF)]

        out_cps = []

        def compute_rows(r0, n):
            y = local_ref[pl.ds(r0, n), :] + comm_ref[pl.ds(r0, n), :]
            rms = jnp.sqrt(jnp.mean(y * y, axis=-1, keepdims=True) + 1e-6)
            stage_ref[pl.ds(r0, n), :] = (y / rms) * gamma_ref[:][None, :]
            cp = pltpu.make_async_copy(
                stage_ref.at[pl.ds(r0, n), :],
                out_ref.at[pl.ds(r0, n), :],
                osem.at[len(out_cps)],
            )
            cp.start()
            out_cps.append(cp)

        for c in range(NCQ):
            zq_send[c].start()
        for c in range(NCD):
            zd_send[c].start()
        local_cp.wait()

        for c in range(NCQ):
            zq_recv[c].wait_recv()
            d_sendR[c].start()
            d_sendL[c].start()
            compute_rows(rows_q(p, c), QCH)

        for c in range(NCQ):
            dL_recv[c].wait_recv()
            if c >= NF:
                f_sendR[c - NF].start()
            compute_rows(rows_q(p - 1, c), QCH)
            dR_recv[c].wait_recv()
            if c < NF:
                f_sendL[c].start()
            compute_rows(rows_q(p + 1, c), QCH)

        for c in range(NCD):
            zd_recv[c].wait_recv()
            compute_rows(c * DCH, DCH)

        for c in range(NF):
            fR_recv[c].wait_recv()
            compute_rows(rows_q(p + 2, c), QCH)
            fL_recv[c].wait_recv()
            compute_rows(rows_q(p + 2, c + NF), QCH)

        for c in range(NCQ):
            zq_send[c].wait_send()
            d_sendR[c].wait_send()
            d_sendL[c].wait_send()
        for c in range(NCD):
            zd_send[c].wait_send()
        for c in range(NF):
            f_sendR[c].wait_send()
            f_sendL[c].wait_send()
        for cp in out_cps:
            cp.wait()

    n_outcp = NCQ + 2 * NCQ + NCD + 2 * NF
    return pl.pallas_call(
        body,
        out_shape=jax.ShapeDtypeStruct((M_HALF, d), jnp.float32),
        in_specs=[
            pl.BlockSpec(memory_space=pl.ANY),
            pl.BlockSpec(memory_space=pltpu.VMEM),
        ],
        out_specs=pl.BlockSpec(memory_space=pl.ANY),
        scratch_shapes=[
            pltpu.VMEM((M_HALF, d), jnp.float32),
            pltpu.VMEM((M_HALF, d), jnp.float32),
            pltpu.VMEM((M_HALF, d), jnp.float32),
            pltpu.SemaphoreType.DMA((NCQ,)),
            pltpu.SemaphoreType.DMA((NCQ,)),
            pltpu.SemaphoreType.DMA((NCD,)),
            pltpu.SemaphoreType.DMA((NCD,)),
            pltpu.SemaphoreType.DMA((NCQ,)),
            pltpu.SemaphoreType.DMA((NCQ,)),
            pltpu.SemaphoreType.DMA((NCQ,)),
            pltpu.SemaphoreType.DMA((NCQ,)),
            pltpu.SemaphoreType.DMA((NF,)),
            pltpu.SemaphoreType.DMA((NF,)),
            pltpu.SemaphoreType.DMA((NF,)),
            pltpu.SemaphoreType.DMA((NF,)),
            pltpu.SemaphoreType.DMA,
            pltpu.SemaphoreType.DMA((n_outcp,)),
        ],
        compiler_params=pltpu.CompilerParams(
            collective_id=0,
            vmem_limit_bytes=56 * 1024 * 1024,
        ),
    )(partial, gamma)
